# baseline (speedup 1.0000x reference)
import functools

import jax
import jax.numpy as jnp
import numpy as np
from jax.experimental import pallas as pl
from jax.experimental.pallas import tpu as pltpu

_HALF_PI = float(np.pi / 2.0)
_ROWS_PER_BLOCK = 4096


def _pair_swap_matrix():
    P = np.zeros((32, 32), np.float32)
    for base in (0, 13):
        for m in range(6):
            a = base + 2 * m
            b = a + 1
            P[b, a] = -1.0
            P[a, b] = 1.0
    return P


def _tc_body(z_ref, ev_ref, P_ref, o_ref):
    x = z_ref[...]
    i = x[:, 12:13]
    ang = i * _HALF_PI
    c0 = jnp.cos(ang)
    s0 = jnp.sin(ang)
    q0 = x[:, 0:1]
    q1 = x[:, 1:2]
    h0 = jnp.abs(c0 * q0 - s0 * q1)
    h1 = jnp.abs(s0 * q0 + c0 * q1)
    h2 = jnp.abs(x[:, 2:3])
    h3 = jnp.abs(x[:, 3:4] + x[:, 4:5] + x[:, 5:6])
    j = jnp.zeros_like(h0)
    m = h0
    j = jnp.where(h1 > m, 1.0, j)
    m = jnp.maximum(m, h1)
    j = jnp.where(h2 > m, 2.0, j)
    m = jnp.maximum(m, h2)
    j = jnp.where(h3 > m, 3.0, j)
    c = jnp.where(j == 0.0, c0, jnp.where(j == 1.0, s0, jnp.where(j == 2.0, -c0, -s0)))
    s = jnp.where(j == 0.0, -s0, jnp.where(j == 1.0, c0, jnp.where(j == 2.0, s0, -c0)))
    P = P_ref[...]
    xP = jax.lax.dot_general(
        x, P, (((1,), (0,)), ((), ())), preferred_element_type=jnp.float32
    )
    y = c * x + s * xP
    lane = jax.lax.broadcasted_iota(jnp.int32, (1, 32), 1)
    y = jnp.where((lane < 25) & (lane != 12), y, x)
    y = jnp.where(lane == 12, j, y)
    ev = ev_ref[...]
    o_ref[...] = jnp.where(ev != 0.0, y, x)


def kernel(t, z, ev):
    del t
    B, D = z.shape
    R = min(_ROWS_PER_BLOCK, B)
    evf = ev.astype(jnp.float32).reshape(B, 1)
    grid = (B // R,)
    return pl.pallas_call(
        _tc_body,
        grid=grid,
        in_specs=[
            pl.BlockSpec((R, D), lambda r: (r, 0)),
            pl.BlockSpec((R, 1), lambda r: (r, 0)),
            pl.BlockSpec((32, 32), lambda r: (0, 0)),
        ],
        out_specs=pl.BlockSpec((R, D), lambda r: (r, 0)),
        out_shape=jax.ShapeDtypeStruct((B, D), jnp.float32),
    )(z, evf, jnp.asarray(_pair_swap_matrix()))


# speedup vs baseline: 2.5279x; 2.5279x over previous
import functools

import jax
import jax.numpy as jnp
import numpy as np
from jax import lax
from jax.experimental import pallas as pl
from jax.experimental.pallas import tpu as pltpu
from jax.experimental.pallas import tpu_sc as plsc

_HALF_PI = float(np.pi / 2.0)
_NC = 2
_NS = 16
_NW = _NC * _NS
_L = 16
_CHUNK = 1024
_PAIRS = [(2 * m, 2 * m + 1) for m in range(6)] + [
    (13 + 2 * m, 14 + 2 * m) for m in range(6)
]


def _sc_group(zbuf, evbuf, g):
    rows = g * _L + lax.iota(jnp.int32, _L)
    rbase = rows * 32

    def col(c):
        return plsc.load_gather(zbuf, [rbase + c])

    ivec = col(12)
    half = jnp.where(ivec >= 0.0, 0.5, -0.5)
    k = (ivec + half).astype(jnp.int32)
    th = (ivec - k.astype(jnp.float32)) * _HALF_PI
    t2 = th * th
    sr = th * (1.0 + t2 * (-1.0 / 6.0 + t2 * (1.0 / 120.0 + t2 * (-1.0 / 5040.0))))
    cr = 1.0 + t2 * (-0.5 + t2 * (1.0 / 24.0 + t2 * (-1.0 / 720.0)))
    q = k & 3
    c0 = jnp.where(q == 0, cr, jnp.where(q == 1, -sr, jnp.where(q == 2, -cr, sr)))
    s0 = jnp.where(q == 0, sr, jnp.where(q == 1, cr, jnp.where(q == 2, -sr, -cr)))
    q0 = col(0)
    q1 = col(1)
    h0 = jnp.abs(c0 * q0 - s0 * q1)
    h1 = jnp.abs(s0 * q0 + c0 * q1)
    h2 = jnp.abs(col(2))
    h3 = jnp.abs(col(3) + col(4) + col(5))
    j = jnp.zeros((_L,), jnp.int32)
    m = h0
    j = jnp.where(h1 > m, 1, j)
    m = jnp.maximum(m, h1)
    j = jnp.where(h2 > m, 2, j)
    m = jnp.maximum(m, h2)
    j = jnp.where(h3 > m, 3, j)
    c = jnp.where(j == 0, c0, jnp.where(j == 1, s0, jnp.where(j == 2, -c0, -s0)))
    s = jnp.where(j == 0, -s0, jnp.where(j == 1, c0, jnp.where(j == 2, s0, -c0)))

    evv = evbuf[pl.ds(g * _L, _L)] != 0

    def put(cidx, vals):
        plsc.store_scatter(zbuf, [rbase + cidx], vals, mask=evv)

    put(12, j.astype(jnp.float32))
    for a, b in _PAIRS:
        xa = q0 if a == 0 else col(a)
        xb = q1 if a == 0 else col(b)
        put(a, c * xa - s * xb)
        put(b, s * xa + c * xb)


def _sc_body(z_hbm, ev_hbm, out_hbm, zbuf, evbuf):
    rows_per_worker = z_hbm.shape[0] // (32 * _NW)
    nchunks = rows_per_worker // _CHUNK
    wid = lax.axis_index("s") * _NC + lax.axis_index("c")
    base = wid * rows_per_worker

    def chunk_body(kc, carry):
        row0 = base + kc * _CHUNK
        pltpu.sync_copy(z_hbm.at[pl.ds(row0 * 32, _CHUNK * 32)], zbuf)
        pltpu.sync_copy(ev_hbm.at[pl.ds(row0, _CHUNK)], evbuf)

        def group_body(g, c2):
            _sc_group(zbuf, evbuf, g)
            return c2

        lax.fori_loop(0, _CHUNK // _L, group_body, 0)
        pltpu.sync_copy(zbuf, out_hbm.at[pl.ds(row0 * 32, _CHUNK * 32)])
        return carry

    lax.fori_loop(0, nchunks, chunk_body, 0)


def kernel(t, z, ev):
    del t
    B, D = z.shape
    ev32 = ev.astype(jnp.int32)
    zflat = z.reshape(B * D)
    run = functools.partial(
        pl.kernel,
        out_type=jax.ShapeDtypeStruct((B * D,), jnp.float32),
        mesh=plsc.VectorSubcoreMesh(core_axis_name="c", subcore_axis_name="s"),
        scratch_types=[
            pltpu.VMEM((_CHUNK * 32,), jnp.float32),
            pltpu.VMEM((_CHUNK,), jnp.int32),
        ],
        compiler_params=pltpu.CompilerParams(needs_layout_passes=False),
    )(_sc_body)
    return run(zflat, ev32).reshape(B, D)


# speedup vs baseline: 2.8552x; 1.1295x over previous
import functools

import jax
import jax.numpy as jnp
import numpy as np
from jax import lax
from jax.experimental import pallas as pl
from jax.experimental.pallas import tpu as pltpu
from jax.experimental.pallas import tpu_sc as plsc

_HALF_PI = float(np.pi / 2.0)
_NC = 2
_NS = 16
_NW = _NC * _NS
_L = 16
_CHUNK = 1024
_PAIRS = [(2 * m, 2 * m + 1) for m in range(6)] + [
    (13 + 2 * m, 14 + 2 * m) for m in range(6)
]


def _sc_group(zbuf, evbuf, g):
    rows = g * _L + lax.iota(jnp.int32, _L)
    rbase = rows * 32

    def col(c):
        return plsc.load_gather(zbuf, [rbase + c])

    ivec = col(12)
    half = jnp.where(ivec >= 0.0, 0.5, -0.5)
    k = (ivec + half).astype(jnp.int32)
    th = (ivec - k.astype(jnp.float32)) * _HALF_PI
    t2 = th * th
    sr = th * (1.0 + t2 * (-1.0 / 6.0 + t2 * (1.0 / 120.0 + t2 * (-1.0 / 5040.0))))
    cr = 1.0 + t2 * (-0.5 + t2 * (1.0 / 24.0 + t2 * (-1.0 / 720.0)))
    q = k & 3
    c0 = jnp.where(q == 0, cr, jnp.where(q == 1, -sr, jnp.where(q == 2, -cr, sr)))
    s0 = jnp.where(q == 0, sr, jnp.where(q == 1, cr, jnp.where(q == 2, -sr, -cr)))
    q0 = col(0)
    q1 = col(1)
    h0 = jnp.abs(c0 * q0 - s0 * q1)
    h1 = jnp.abs(s0 * q0 + c0 * q1)
    h2 = jnp.abs(col(2))
    h3 = jnp.abs(col(3) + col(4) + col(5))
    j = jnp.zeros((_L,), jnp.int32)
    m = h0
    j = jnp.where(h1 > m, 1, j)
    m = jnp.maximum(m, h1)
    j = jnp.where(h2 > m, 2, j)
    m = jnp.maximum(m, h2)
    j = jnp.where(h3 > m, 3, j)
    c = jnp.where(j == 0, c0, jnp.where(j == 1, s0, jnp.where(j == 2, -c0, -s0)))
    s = jnp.where(j == 0, -s0, jnp.where(j == 1, c0, jnp.where(j == 2, s0, -c0)))

    evv = evbuf[pl.ds(g * _L, _L)] != 0

    def put(cidx, vals):
        plsc.store_scatter(zbuf, [rbase + cidx], vals, mask=evv)

    put(12, j.astype(jnp.float32))
    for a, b in _PAIRS:
        xa = q0 if a == 0 else col(a)
        xb = q1 if a == 0 else col(b)
        put(a, c * xa - s * xb)
        put(b, s * xa + c * xb)


def _sc_body(z_hbm, ev_hbm, out_hbm, zbuf, evbuf):
    rows_per_worker = z_hbm.shape[0] // (32 * _NW)
    nchunks = rows_per_worker // _CHUNK
    wid = lax.axis_index("s") * _NC + lax.axis_index("c")
    base = wid * rows_per_worker

    def chunk_body(kc, carry):
        row0 = base + kc * _CHUNK
        pltpu.sync_copy(z_hbm.at[pl.ds(row0 * 32, _CHUNK * 32)], zbuf)
        pltpu.sync_copy(ev_hbm.at[pl.ds(row0, _CHUNK)], evbuf)

        @plsc.parallel_loop(0, _CHUNK // _L, unroll=2)
        def group_body(g):
            _sc_group(zbuf, evbuf, g)
        pltpu.sync_copy(zbuf, out_hbm.at[pl.ds(row0 * 32, _CHUNK * 32)])
        return carry

    lax.fori_loop(0, nchunks, chunk_body, 0)


def kernel(t, z, ev):
    del t
    B, D = z.shape
    ev32 = ev.astype(jnp.int32)
    zflat = z.reshape(B * D)
    run = functools.partial(
        pl.kernel,
        out_type=jax.ShapeDtypeStruct((B * D,), jnp.float32),
        mesh=plsc.VectorSubcoreMesh(core_axis_name="c", subcore_axis_name="s"),
        scratch_types=[
            pltpu.VMEM((_CHUNK * 32,), jnp.float32),
            pltpu.VMEM((_CHUNK,), jnp.int32),
        ],
        compiler_params=pltpu.CompilerParams(needs_layout_passes=False),
    )(_sc_body)
    return run(zflat, ev32).reshape(B, D)
